# initial kernel scaffold (unmeasured)
import jax
import jax.numpy as jnp
from jax import lax
from jax.experimental import pallas as pl
from jax.experimental.pallas import tpu as pltpu

N_DEV = 8


def kernel(x, w_mat, scale_x, scale_w):
    m, k = x.shape
    _, n = w_mat.shape
    ch = m // N_DEV

    def body(x_ref, w_ref, sx_ref, sw_ref, out_ref,
             wb_ref, acc_ref, recv_ref,
             send_sem, recv_sem, copy_sem, credit_sem,
             ag_send_sems, ag_recv_sems):
        d = lax.axis_index("i")
        left = lax.rem(d - 1 + N_DEV, N_DEV)
        right = lax.rem(d + 1, N_DEV)

        barrier = pltpu.get_barrier_semaphore()
        for o in range(1, N_DEV):
            pl.semaphore_signal(
                barrier, inc=1,
                device_id=(lax.rem(d + o, N_DEV),),
                device_id_type=pl.DeviceIdType.MESH)
        pl.semaphore_wait(barrier, N_DEV - 1)

        wb_ref[...] = w_ref[...].astype(jnp.bfloat16)

        def partial(c):
            xa = x_ref[pl.ds(c * ch, ch), :].astype(jnp.bfloat16)
            return lax.dot_general(
                xa, wb_ref[...], (((1,), (0,)), ((), ())),
                preferred_element_type=jnp.float32)

        acc_ref[...] = partial(d)
        for s in range(N_DEV - 1):
            if s >= 1:
                pl.semaphore_wait(credit_sem, 1)
            rdma = pltpu.make_async_remote_copy(
                src_ref=acc_ref, dst_ref=recv_ref,
                send_sem=send_sem, recv_sem=recv_sem,
                device_id=(right,), device_id_type=pl.DeviceIdType.MESH)
            rdma.start()
            p = partial(lax.rem(d - s - 1 + N_DEV, N_DEV))
            rdma.wait()
            acc_ref[...] = p + recv_ref[...]
            pl.semaphore_signal(credit_sem, inc=1, device_id=(left,),
                                device_id_type=pl.DeviceIdType.MESH)

        acc_ref[...] = acc_ref[...] * (sx_ref[0] * sw_ref[0])

        own = lax.rem(d + 1, N_DEV)
        cp = pltpu.make_async_copy(
            acc_ref, out_ref.at[pl.ds(own * ch, ch), :], copy_sem)
        cp.start()
        cp.wait()

        rdmas = []
        for o in range(1, N_DEV):
            rd = pltpu.make_async_remote_copy(
                src_ref=acc_ref,
                dst_ref=out_ref.at[pl.ds(own * ch, ch), :],
                send_sem=ag_send_sems.at[o - 1],
                recv_sem=ag_recv_sems.at[o - 1],
                device_id=(lax.rem(d + o, N_DEV),),
                device_id_type=pl.DeviceIdType.MESH)
            rd.start()
            rdmas.append(rd)
        for rd in rdmas:
            rd.wait_send()
        for rd in rdmas:
            rd.wait_recv()

    return pl.pallas_call(
        body,
        out_shape=jax.ShapeDtypeStruct((m, n), jnp.float32),
        in_specs=[
            pl.BlockSpec(memory_space=pltpu.VMEM),
            pl.BlockSpec(memory_space=pltpu.VMEM),
            pl.BlockSpec(memory_space=pltpu.SMEM),
            pl.BlockSpec(memory_space=pltpu.SMEM),
        ],
        out_specs=pl.BlockSpec(memory_space=pltpu.ANY),
        scratch_shapes=[
            pltpu.VMEM((k, n), jnp.bfloat16),
            pltpu.VMEM((ch, n), jnp.float32),
            pltpu.VMEM((ch, n), jnp.float32),
            pltpu.SemaphoreType.DMA,
            pltpu.SemaphoreType.DMA,
            pltpu.SemaphoreType.DMA,
            pltpu.SemaphoreType.REGULAR,
            pltpu.SemaphoreType.DMA((N_DEV - 1,)),
            pltpu.SemaphoreType.DMA((N_DEV - 1,)),
        ],
        compiler_params=pltpu.CompilerParams(collective_id=0),
    )(x, w_mat, scale_x, scale_w)


# baseline (device time: 1282297 ns/iter reference)
import os

import jax
import jax.numpy as jnp
from jax import lax
from jax.experimental import pallas as pl
from jax.experimental.pallas import tpu as pltpu

N_DEV = 8
_PHASE = int(os.environ.get("KPHASE", "3"))
_CREDIT = int(os.environ.get("KCREDIT", "1"))
NQ = 4


def kernel(x, w_mat, scale_x, scale_w):
    m, k = x.shape
    _, n = w_mat.shape
    ch = m // N_DEV
    nq = n // NQ

    def body(x_hbm, w_hbm, sx_ref, sw_ref, out_ref,
             xb_ref, wb_ref, acc_ref, recv_ref, pbuf_ref,
             stage_x, stage_w, tok_src, tok_dst,
             send_sem, recv_sem, copy_sem, tok_send_sem, tok_recv_sem,
             ag_send_sems, ag_recv_sems):
        d = lax.axis_index("i")
        left = lax.rem(d - 1 + N_DEV, N_DEV)
        right = lax.rem(d + 1, N_DEV)

        if _PHASE >= 1:
            barrier = pltpu.get_barrier_semaphore()
            for o in range(1, N_DEV):
                pl.semaphore_signal(
                    barrier, inc=1,
                    device_id=(lax.rem(d + o, N_DEV),),
                    device_id_type=pl.DeviceIdType.MESH)
            pl.semaphore_wait(barrier, N_DEV - 1)

        for i in range(m // ch):
            cp = pltpu.make_async_copy(
                x_hbm.at[pl.ds(i * ch, ch), :], stage_x, copy_sem)
            cp.start()
            cp.wait()
            xb_ref[pl.ds(i * ch, ch), :] = stage_x[...].astype(jnp.bfloat16)
        for j in range(NQ):
            cp = pltpu.make_async_copy(
                w_hbm.at[:, pl.ds(j * nq, nq)], stage_w, copy_sem)
            cp.start()
            cp.wait()
            wb_ref[:, pl.ds(j * nq, nq)] = stage_w[...].astype(jnp.bfloat16)

        def partial_into(dst_ref, c, add_ref=None):
            xa = xb_ref[pl.ds(c * ch, ch), :]
            for q in range(NQ):
                p = lax.dot_general(
                    xa, wb_ref[:, pl.ds(q * nq, nq)],
                    (((1,), (0,)), ((), ())),
                    preferred_element_type=jnp.float32)
                if add_ref is not None:
                    p = p + add_ref[:, pl.ds(q * nq, nq)].astype(jnp.float32)
                dst_ref[:, pl.ds(q * nq, nq)] = p.astype(jnp.bfloat16)

        def token():
            return pltpu.make_async_remote_copy(
                src_ref=tok_src, dst_ref=tok_dst,
                send_sem=tok_send_sem, recv_sem=tok_recv_sem,
                device_id=(left,), device_id_type=pl.DeviceIdType.MESH)

        partial_into(acc_ref, d)
        for s in range(N_DEV - 1 if _PHASE >= 2 else 0):
            if s >= 1 and _CREDIT:
                tok = token()
                tok.wait_recv()
            rdma = pltpu.make_async_remote_copy(
                src_ref=acc_ref, dst_ref=recv_ref,
                send_sem=send_sem, recv_sem=recv_sem,
                device_id=(right,), device_id_type=pl.DeviceIdType.MESH)
            rdma.start()
            partial_into(pbuf_ref, lax.rem(d - s - 1 + N_DEV, N_DEV))
            rdma.wait()
            acc_ref[...] = (pbuf_ref[...].astype(jnp.float32)
                            + recv_ref[...].astype(jnp.float32)
                            ).astype(jnp.bfloat16)
            if _CREDIT and s < N_DEV - 2:
                tok = token()
                tok.start()
                tok.wait_send()

        scale = sx_ref[0] * sw_ref[0]
        acc_ref[...] = (acc_ref[...].astype(jnp.float32) * scale
                        ).astype(jnp.bfloat16)

        own = lax.rem(d + 1, N_DEV)
        cp = pltpu.make_async_copy(
            acc_ref, out_ref.at[pl.ds(own * ch, ch), :], copy_sem)
        cp.start()
        cp.wait()

        rdmas = []
        for o in range(1, N_DEV if _PHASE >= 3 else 1):
            rd = pltpu.make_async_remote_copy(
                src_ref=acc_ref,
                dst_ref=out_ref.at[pl.ds(own * ch, ch), :],
                send_sem=ag_send_sems.at[o - 1],
                recv_sem=ag_recv_sems.at[o - 1],
                device_id=(lax.rem(d + o, N_DEV),),
                device_id_type=pl.DeviceIdType.MESH)
            rd.start()
            rdmas.append(rd)
        for rd in rdmas:
            rd.wait_send()
        for rd in rdmas:
            rd.wait_recv()

    return pl.pallas_call(
        body,
        out_shape=jax.ShapeDtypeStruct((m, n), jnp.bfloat16),
        in_specs=[
            pl.BlockSpec(memory_space=pl.ANY),
            pl.BlockSpec(memory_space=pl.ANY),
            pl.BlockSpec(memory_space=pltpu.SMEM),
            pl.BlockSpec(memory_space=pltpu.SMEM),
        ],
        out_specs=pl.BlockSpec(memory_space=pl.ANY),
        scratch_shapes=[
            pltpu.VMEM((m, k), jnp.bfloat16),
            pltpu.VMEM((k, n), jnp.bfloat16),
            pltpu.VMEM((ch, n), jnp.bfloat16),
            pltpu.VMEM((ch, n), jnp.bfloat16),
            pltpu.VMEM((ch, n), jnp.bfloat16),
            pltpu.VMEM((ch, k), jnp.float32),
            pltpu.VMEM((k, n // NQ), jnp.float32),
            pltpu.VMEM((8, 128), jnp.float32),
            pltpu.VMEM((8, 128), jnp.float32),
            pltpu.SemaphoreType.DMA,
            pltpu.SemaphoreType.DMA,
            pltpu.SemaphoreType.DMA,
            pltpu.SemaphoreType.DMA,
            pltpu.SemaphoreType.DMA,
            pltpu.SemaphoreType.DMA((N_DEV - 1,)),
            pltpu.SemaphoreType.DMA((N_DEV - 1,)),
        ],
        compiler_params=pltpu.CompilerParams(
            vmem_limit_bytes=100 * 1024 * 1024,
            **({"collective_id": 0} if _PHASE >= 1 else {})),
    )(x, w_mat, scale_x, scale_w)


# device time: 795450 ns/iter; 1.6120x vs baseline; 1.6120x over previous
import jax
import jax.numpy as jnp
from jax import lax
from jax.experimental import pallas as pl
from jax.experimental.pallas import tpu as pltpu

N_DEV = 8


def kernel(x, w_mat, scale_x, scale_w):
    m, k = x.shape
    _, n = w_mat.shape
    ch = m // N_DEV
    hn = n // 2
    qn = hn // 2

    def body(x_hbm, w_hbm, sx_ref, sw_ref, out_ref,
             xb_ref, wb_ref, accA, accB, bufA0, bufA1, bufB0, bufB1,
             stage_x, stage_w, tok_src, tok_dst,
             rsA_send, rsA_recv, rsB_send, rsB_recv,
             agA_send, agA_recv, agB_send, agB_recv,
             rtokA_s, rtokA_r, rtokB_s, rtokB_r,
             atokA_s, atokA_r, atokB_s, atokB_r,
             copy_semA, copy_semB):
        d = lax.axis_index("i")
        left = lax.rem(d - 1 + N_DEV, N_DEV)
        right = lax.rem(d + 1, N_DEV)

        barrier = pltpu.get_barrier_semaphore()
        for o in range(1, N_DEV):
            pl.semaphore_signal(
                barrier, inc=1,
                device_id=(lax.rem(d + o, N_DEV),),
                device_id_type=pl.DeviceIdType.MESH)
        pl.semaphore_wait(barrier, N_DEV - 1)

        for i in range(m // ch):
            cp = pltpu.make_async_copy(
                x_hbm.at[pl.ds(i * ch, ch), :], stage_x, copy_semA)
            cp.start()
            cp.wait()
            xb_ref[pl.ds(i * ch, ch), :] = stage_x[...].astype(jnp.bfloat16)
        for j in range(n // qn):
            cp = pltpu.make_async_copy(
                w_hbm.at[:, pl.ds(j * qn, qn)], stage_w, copy_semA)
            cp.start()
            cp.wait()
            wb_ref[:, pl.ds(j * qn, qn)] = stage_w[...].astype(jnp.bfloat16)

        def partial_half(dst_ref, c, half, add_ref=None):
            xa = xb_ref[pl.ds(c * ch, ch), :]
            for q in range(2):
                p = lax.dot_general(
                    xa, wb_ref[:, pl.ds(half * hn + q * qn, qn)],
                    (((1,), (0,)), ((), ())),
                    preferred_element_type=jnp.float32)
                if add_ref is not None:
                    p = p + add_ref[:, pl.ds(q * qn, qn)].astype(jnp.float32)
                dst_ref[:, pl.ds(q * qn, qn)] = p.astype(jnp.bfloat16)

        def remote(src, dst, ssem, rsem, dev):
            return pltpu.make_async_remote_copy(
                src_ref=src, dst_ref=dst, send_sem=ssem, recv_sem=rsem,
                device_id=(dev,), device_id_type=pl.DeviceIdType.MESH)

        def token(ssem, rsem, dev):
            return remote(tok_src, tok_dst, ssem, rsem, dev)

        partial_half(accA, d, 0)
        partial_half(accB, d, 1)
        for s in range(N_DEV - 1):
            if s >= 1:
                token(rtokA_s, rtokA_r, left).wait_recv()
                token(rtokB_s, rtokB_r, right).wait_recv()
            rdA = remote(accA, bufA0, rsA_send, rsA_recv, right)
            rdB = remote(accB, bufB0, rsB_send, rsB_recv, left)
            rdA.start()
            rdB.start()
            partial_half(bufA1, lax.rem(d - s - 1 + N_DEV, N_DEV), 0)
            partial_half(bufB1, lax.rem(d + s + 1, N_DEV), 1)
            rdA.wait()
            rdB.wait()
            accA[...] = (bufA1[...].astype(jnp.float32)
                         + bufA0[...].astype(jnp.float32)).astype(jnp.bfloat16)
            accB[...] = (bufB1[...].astype(jnp.float32)
                         + bufB0[...].astype(jnp.float32)).astype(jnp.bfloat16)
            if s < N_DEV - 2:
                tA = token(rtokA_s, rtokA_r, left)
                tB = token(rtokB_s, rtokB_r, right)
                tA.start()
                tB.start()
                tA.wait_send()
                tB.wait_send()

        scale = sx_ref[0] * sw_ref[0]
        accA[...] = (accA[...].astype(jnp.float32) * scale).astype(jnp.bfloat16)
        accB[...] = (accB[...].astype(jnp.float32) * scale).astype(jnp.bfloat16)

        ownA = lax.rem(d + 1, N_DEV)
        ownB = lax.rem(d - 1 + N_DEV, N_DEV)
        cpA = pltpu.make_async_copy(
            accA, out_ref.at[pl.ds(ownA * ch, ch), pl.ds(0, hn)], copy_semA)
        cpB = pltpu.make_async_copy(
            accB, out_ref.at[pl.ds(ownB * ch, ch), pl.ds(hn, hn)], copy_semB)
        cpA.start()
        cpB.start()
        cpA.wait()
        cpB.wait()

        slotsA = (bufA0, bufA1)
        slotsB = (bufB0, bufB1)
        for t in range(N_DEV - 1):
            if t >= 2:
                token(atokA_s, atokA_r, left).wait_recv()
                token(atokB_s, atokB_r, right).wait_recv()
            srcA = accA if t == 0 else slotsA[(t - 1) % 2]
            srcB = accB if t == 0 else slotsB[(t - 1) % 2]
            rdA = remote(srcA, slotsA[t % 2],
                         agA_send.at[t % 2], agA_recv.at[t % 2], right)
            rdB = remote(srcB, slotsB[t % 2],
                         agB_send.at[t % 2], agB_recv.at[t % 2], left)
            rdA.start()
            rdB.start()
            rdA.wait()
            rdB.wait()
            rowA = lax.rem(d - t + N_DEV, N_DEV)
            rowB = lax.rem(d + t, N_DEV)
            cpA = pltpu.make_async_copy(
                slotsA[t % 2],
                out_ref.at[pl.ds(rowA * ch, ch), pl.ds(0, hn)], copy_semA)
            cpB = pltpu.make_async_copy(
                slotsB[t % 2],
                out_ref.at[pl.ds(rowB * ch, ch), pl.ds(hn, hn)], copy_semB)
            cpA.start()
            cpB.start()
            cpA.wait()
            cpB.wait()
            if 1 <= t <= N_DEV - 3:
                tA = token(atokA_s, atokA_r, left)
                tB = token(atokB_s, atokB_r, right)
                tA.start()
                tB.start()
                tA.wait_send()
                tB.wait_send()

    return pl.pallas_call(
        body,
        out_shape=jax.ShapeDtypeStruct((m, n), jnp.bfloat16),
        in_specs=[
            pl.BlockSpec(memory_space=pl.ANY),
            pl.BlockSpec(memory_space=pl.ANY),
            pl.BlockSpec(memory_space=pltpu.SMEM),
            pl.BlockSpec(memory_space=pltpu.SMEM),
        ],
        out_specs=pl.BlockSpec(memory_space=pl.ANY),
        scratch_shapes=[
            pltpu.VMEM((m, k), jnp.bfloat16),
            pltpu.VMEM((k, n), jnp.bfloat16),
            pltpu.VMEM((ch, hn), jnp.bfloat16),
            pltpu.VMEM((ch, hn), jnp.bfloat16),
            pltpu.VMEM((ch, hn), jnp.bfloat16),
            pltpu.VMEM((ch, hn), jnp.bfloat16),
            pltpu.VMEM((ch, hn), jnp.bfloat16),
            pltpu.VMEM((ch, hn), jnp.bfloat16),
            pltpu.VMEM((ch, k), jnp.float32),
            pltpu.VMEM((k, qn), jnp.float32),
            pltpu.VMEM((8, 128), jnp.float32),
            pltpu.VMEM((8, 128), jnp.float32),
            pltpu.SemaphoreType.DMA,
            pltpu.SemaphoreType.DMA,
            pltpu.SemaphoreType.DMA,
            pltpu.SemaphoreType.DMA,
            pltpu.SemaphoreType.DMA((2,)),
            pltpu.SemaphoreType.DMA((2,)),
            pltpu.SemaphoreType.DMA((2,)),
            pltpu.SemaphoreType.DMA((2,)),
            pltpu.SemaphoreType.DMA,
            pltpu.SemaphoreType.DMA,
            pltpu.SemaphoreType.DMA,
            pltpu.SemaphoreType.DMA,
            pltpu.SemaphoreType.DMA,
            pltpu.SemaphoreType.DMA,
            pltpu.SemaphoreType.DMA,
            pltpu.SemaphoreType.DMA,
            pltpu.SemaphoreType.DMA,
            pltpu.SemaphoreType.DMA,
        ],
        compiler_params=pltpu.CompilerParams(
            collective_id=0, vmem_limit_bytes=100 * 1024 * 1024),
    )(x, w_mat, scale_x, scale_w)


# device time: 758685 ns/iter; 1.6902x vs baseline; 1.0485x over previous
import jax
import jax.numpy as jnp
from jax import lax
from jax.experimental import pallas as pl
from jax.experimental.pallas import tpu as pltpu

N_DEV = 8
P = 2


def kernel(x, w_mat, scale_x, scale_w):
    m, k = x.shape
    _, n = w_mat.shape
    ch = m // N_DEV
    hn = n // 2
    qn = hn // 2

    def body(x_hbm, w_hbm, sx_ref, sw_ref, out_ref,
             xb_ref, wb_ref, accA, accB, bufA0, bufA1, bufB0, bufB1,
             stage_x0, stage_x1, stage_w0, stage_w1, tok_src, tok_dst,
             rsA_send, rsA_recv, rsB_send, rsB_recv,
             agA_send, agA_recv, agB_send, agB_recv,
             rtokA_s, rtokA_r, rtokB_s, rtokB_r,
             atokA_s, atokA_r, atokB_s, atokB_r,
             copyA_sems, copyB_sems, own_semA, own_semB, stg_sems):
        d = lax.axis_index("i")
        left = lax.rem(d - 1 + N_DEV, N_DEV)
        right = lax.rem(d + 1, N_DEV)

        barrier = pltpu.get_barrier_semaphore()
        for o in range(1, N_DEV):
            pl.semaphore_signal(
                barrier, inc=1,
                device_id=(lax.rem(d + o, N_DEV),),
                device_id_type=pl.DeviceIdType.MESH)
        pl.semaphore_wait(barrier, N_DEV - 1)

        sx_stages = (stage_x0, stage_x1)
        prev = None
        for i in range(m // ch):
            cp = pltpu.make_async_copy(
                x_hbm.at[pl.ds(i * ch, ch), :], sx_stages[i % 2],
                stg_sems.at[i % 2])
            cp.start()
            if prev is not None:
                j, pcp = prev
                pcp.wait()
                xb_ref[pl.ds(j * ch, ch), :] = (
                    sx_stages[j % 2][...].astype(jnp.bfloat16))
            prev = (i, cp)
        j, pcp = prev
        pcp.wait()
        xb_ref[pl.ds(j * ch, ch), :] = sx_stages[j % 2][...].astype(jnp.bfloat16)

        sw_stages = (stage_w0, stage_w1)
        prev = None
        for i in range(n // qn):
            cp = pltpu.make_async_copy(
                w_hbm.at[:, pl.ds(i * qn, qn)], sw_stages[i % 2],
                stg_sems.at[i % 2])
            cp.start()
            if prev is not None:
                j, pcp = prev
                pcp.wait()
                wb_ref[:, pl.ds(j * qn, qn)] = (
                    sw_stages[j % 2][...].astype(jnp.bfloat16))
            prev = (i, cp)
        j, pcp = prev
        pcp.wait()
        wb_ref[:, pl.ds(j * qn, qn)] = sw_stages[j % 2][...].astype(jnp.bfloat16)

        def partial_half(dst_ref, c, half, add_ref=None):
            xa = xb_ref[pl.ds(c * ch, ch), :]
            for q in range(2):
                p = lax.dot_general(
                    xa, wb_ref[:, pl.ds(half * hn + q * qn, qn)],
                    (((1,), (0,)), ((), ())),
                    preferred_element_type=jnp.float32)
                if add_ref is not None:
                    p = p + add_ref[:, pl.ds(q * qn, qn)].astype(jnp.float32)
                dst_ref[:, pl.ds(q * qn, qn)] = p.astype(jnp.bfloat16)

        def remote(src, dst, ssem, rsem, dev):
            return pltpu.make_async_remote_copy(
                src_ref=src, dst_ref=dst, send_sem=ssem, recv_sem=rsem,
                device_id=(dev,), device_id_type=pl.DeviceIdType.MESH)

        def token(ssem, rsem, dev):
            return remote(tok_src, tok_dst, ssem, rsem, dev)

        partial_half(accA, d, 0)
        partial_half(accB, d, 1)
        for s in range(N_DEV - 1):
            if s >= 1:
                token(rtokA_s, rtokA_r, left).wait_recv()
                token(rtokB_s, rtokB_r, right).wait_recv()
            rdsA, rdsB = [], []
            for pc in range(P):
                cs = pl.ds(pc * qn, qn)
                rdsA.append(remote(accA.at[:, cs], bufA0.at[:, cs],
                                   rsA_send.at[pc], rsA_recv.at[pc], right))
                rdsB.append(remote(accB.at[:, cs], bufB0.at[:, cs],
                                   rsB_send.at[pc], rsB_recv.at[pc], left))
            for r in rdsA:
                r.start()
            for r in rdsB:
                r.start()
            partial_half(bufA1, lax.rem(d - s - 1 + N_DEV, N_DEV), 0)
            partial_half(bufB1, lax.rem(d + s + 1, N_DEV), 1)
            for pc in range(P):
                cs = pl.ds(pc * qn, qn)
                rdsA[pc].wait()
                accA[:, cs] = (bufA1[:, cs].astype(jnp.float32)
                               + bufA0[:, cs].astype(jnp.float32)
                               ).astype(jnp.bfloat16)
                rdsB[pc].wait()
                accB[:, cs] = (bufB1[:, cs].astype(jnp.float32)
                               + bufB0[:, cs].astype(jnp.float32)
                               ).astype(jnp.bfloat16)
            if s < N_DEV - 2:
                tA = token(rtokA_s, rtokA_r, left)
                tB = token(rtokB_s, rtokB_r, right)
                tA.start()
                tB.start()
                tA.wait_send()
                tB.wait_send()

        scale = sx_ref[0] * sw_ref[0]
        accA[...] = (accA[...].astype(jnp.float32) * scale).astype(jnp.bfloat16)
        accB[...] = (accB[...].astype(jnp.float32) * scale).astype(jnp.bfloat16)

        ownA = lax.rem(d + 1, N_DEV)
        ownB = lax.rem(d - 1 + N_DEV, N_DEV)
        cpA = pltpu.make_async_copy(
            accA, out_ref.at[pl.ds(ownA * ch, ch), pl.ds(0, hn)], own_semA)
        cpB = pltpu.make_async_copy(
            accB, out_ref.at[pl.ds(ownB * ch, ch), pl.ds(hn, hn)], own_semB)
        cpA.start()
        cpB.start()

        slotsA = (bufA0, bufA1)
        slotsB = (bufB0, bufB1)
        out_cps = {}
        for t in range(N_DEV - 1):
            if t >= 2:
                token(atokA_s, atokA_r, left).wait_recv()
                token(atokB_s, atokB_r, right).wait_recv()
            srcA = accA if t == 0 else slotsA[(t - 1) % 2]
            srcB = accB if t == 0 else slotsB[(t - 1) % 2]
            rdA = remote(srcA, slotsA[t % 2],
                         agA_send.at[t % 2], agA_recv.at[t % 2], right)
            rdB = remote(srcB, slotsB[t % 2],
                         agB_send.at[t % 2], agB_recv.at[t % 2], left)
            rdA.start()
            rdB.start()
            rdA.wait()
            rdB.wait()
            rowA = lax.rem(d - t + N_DEV, N_DEV)
            rowB = lax.rem(d + t, N_DEV)
            cpA = pltpu.make_async_copy(
                slotsA[t % 2],
                out_ref.at[pl.ds(rowA * ch, ch), pl.ds(0, hn)],
                copyA_sems.at[t % 2])
            cpB = pltpu.make_async_copy(
                slotsB[t % 2],
                out_ref.at[pl.ds(rowB * ch, ch), pl.ds(hn, hn)],
                copyB_sems.at[t % 2])
            cpA.start()
            cpB.start()
            out_cps[t % 2] = (cpA, cpB)
            if 1 <= t <= N_DEV - 3:
                pA, pB = out_cps[(t - 1) % 2]
                pA.wait()
                pB.wait()
                tA = token(atokA_s, atokA_r, left)
                tB = token(atokB_s, atokB_r, right)
                tA.start()
                tB.start()
                tA.wait_send()
                tB.wait_send()

        for slot in ((N_DEV - 3) % 2, (N_DEV - 2) % 2):
            pA, pB = out_cps[slot]
            pA.wait()
            pB.wait()
        pltpu.make_async_copy(
            accA, out_ref.at[pl.ds(ownA * ch, ch), pl.ds(0, hn)],
            own_semA).wait()
        pltpu.make_async_copy(
            accB, out_ref.at[pl.ds(ownB * ch, ch), pl.ds(hn, hn)],
            own_semB).wait()

    return pl.pallas_call(
        body,
        out_shape=jax.ShapeDtypeStruct((m, n), jnp.bfloat16),
        in_specs=[
            pl.BlockSpec(memory_space=pl.ANY),
            pl.BlockSpec(memory_space=pl.ANY),
            pl.BlockSpec(memory_space=pltpu.SMEM),
            pl.BlockSpec(memory_space=pltpu.SMEM),
        ],
        out_specs=pl.BlockSpec(memory_space=pl.ANY),
        scratch_shapes=[
            pltpu.VMEM((m, k), jnp.bfloat16),
            pltpu.VMEM((k, n), jnp.bfloat16),
            pltpu.VMEM((ch, hn), jnp.bfloat16),
            pltpu.VMEM((ch, hn), jnp.bfloat16),
            pltpu.VMEM((ch, hn), jnp.bfloat16),
            pltpu.VMEM((ch, hn), jnp.bfloat16),
            pltpu.VMEM((ch, hn), jnp.bfloat16),
            pltpu.VMEM((ch, hn), jnp.bfloat16),
            pltpu.VMEM((ch, k), jnp.float32),
            pltpu.VMEM((ch, k), jnp.float32),
            pltpu.VMEM((k, qn), jnp.float32),
            pltpu.VMEM((k, qn), jnp.float32),
            pltpu.VMEM((8, 128), jnp.float32),
            pltpu.VMEM((8, 128), jnp.float32),
            pltpu.SemaphoreType.DMA((P,)),
            pltpu.SemaphoreType.DMA((P,)),
            pltpu.SemaphoreType.DMA((P,)),
            pltpu.SemaphoreType.DMA((P,)),
            pltpu.SemaphoreType.DMA((2,)),
            pltpu.SemaphoreType.DMA((2,)),
            pltpu.SemaphoreType.DMA((2,)),
            pltpu.SemaphoreType.DMA((2,)),
            pltpu.SemaphoreType.DMA,
            pltpu.SemaphoreType.DMA,
            pltpu.SemaphoreType.DMA,
            pltpu.SemaphoreType.DMA,
            pltpu.SemaphoreType.DMA,
            pltpu.SemaphoreType.DMA,
            pltpu.SemaphoreType.DMA,
            pltpu.SemaphoreType.DMA,
            pltpu.SemaphoreType.DMA((2,)),
            pltpu.SemaphoreType.DMA((2,)),
            pltpu.SemaphoreType.DMA,
            pltpu.SemaphoreType.DMA,
            pltpu.SemaphoreType.DMA((2,)),
        ],
        compiler_params=pltpu.CompilerParams(
            collective_id=0, vmem_limit_bytes=100 * 1024 * 1024),
    )(x, w_mat, scale_x, scale_w)


# device time: 754041 ns/iter; 1.7006x vs baseline; 1.0062x over previous
import jax
import jax.numpy as jnp
from jax import lax
from jax.experimental import pallas as pl
from jax.experimental.pallas import tpu as pltpu

N_DEV = 8
P = 4


def kernel(x, w_mat, scale_x, scale_w):
    m, k = x.shape
    _, n = w_mat.shape
    ch = m // N_DEV
    hn = n // 2
    qn = hn // 2
    pn = hn // P

    def body(x_hbm, w_hbm, sx_ref, sw_ref, out_ref,
             xb_ref, wb_ref, accA, accB, bufA0, bufA1, bufB0, bufB1,
             stage_x0, stage_x1, stage_w0, stage_w1, tok_src, tok_dst,
             rsA_send, rsA_recv, rsB_send, rsB_recv,
             agA_send, agA_recv, agB_send, agB_recv,
             rtokA_s, rtokA_r, rtokB_s, rtokB_r,
             atokA_s, atokA_r, atokB_s, atokB_r,
             copyA_sems, copyB_sems, own_semA, own_semB, stg_sems):
        d = lax.axis_index("i")
        left = lax.rem(d - 1 + N_DEV, N_DEV)
        right = lax.rem(d + 1, N_DEV)

        barrier = pltpu.get_barrier_semaphore()
        for o in range(1, N_DEV):
            pl.semaphore_signal(
                barrier, inc=1,
                device_id=(lax.rem(d + o, N_DEV),),
                device_id_type=pl.DeviceIdType.MESH)
        pl.semaphore_wait(barrier, N_DEV - 1)

        sx_stages = (stage_x0, stage_x1)
        prev = None
        for i in range(m // ch):
            cp = pltpu.make_async_copy(
                x_hbm.at[pl.ds(i * ch, ch), :], sx_stages[i % 2],
                stg_sems.at[i % 2])
            cp.start()
            if prev is not None:
                j, pcp = prev
                pcp.wait()
                xb_ref[pl.ds(j * ch, ch), :] = (
                    sx_stages[j % 2][...].astype(jnp.bfloat16))
            prev = (i, cp)
        j, pcp = prev
        pcp.wait()
        xb_ref[pl.ds(j * ch, ch), :] = sx_stages[j % 2][...].astype(jnp.bfloat16)

        sw_stages = (stage_w0, stage_w1)
        prev = None
        for i in range(n // qn):
            cp = pltpu.make_async_copy(
                w_hbm.at[:, pl.ds(i * qn, qn)], sw_stages[i % 2],
                stg_sems.at[i % 2])
            cp.start()
            if prev is not None:
                j, pcp = prev
                pcp.wait()
                wb_ref[:, pl.ds(j * qn, qn)] = (
                    sw_stages[j % 2][...].astype(jnp.bfloat16))
            prev = (i, cp)
        j, pcp = prev
        pcp.wait()
        wb_ref[:, pl.ds(j * qn, qn)] = sw_stages[j % 2][...].astype(jnp.bfloat16)

        def partial_half(dst_ref, c, half, add_ref=None):
            xa = xb_ref[pl.ds(c * ch, ch), :]
            for q in range(2):
                p = lax.dot_general(
                    xa, wb_ref[:, pl.ds(half * hn + q * qn, qn)],
                    (((1,), (0,)), ((), ())),
                    preferred_element_type=jnp.float32)
                if add_ref is not None:
                    p = p + add_ref[:, pl.ds(q * qn, qn)].astype(jnp.float32)
                dst_ref[:, pl.ds(q * qn, qn)] = p.astype(jnp.bfloat16)

        def remote(src, dst, ssem, rsem, dev):
            return pltpu.make_async_remote_copy(
                src_ref=src, dst_ref=dst, send_sem=ssem, recv_sem=rsem,
                device_id=(dev,), device_id_type=pl.DeviceIdType.MESH)

        def token(ssem, rsem, dev):
            return remote(tok_src, tok_dst, ssem, rsem, dev)

        scale = sx_ref[0] * sw_ref[0]
        partial_half(accA, d, 0)
        partial_half(accB, d, 1)
        for s in range(N_DEV - 1):
            if s >= 1:
                token(rtokA_s, rtokA_r, left).wait_recv()
                token(rtokB_s, rtokB_r, right).wait_recv()
            rdsA, rdsB = [], []
            for pc in range(P):
                cs = pl.ds(pc * pn, pn)
                rdsA.append(remote(accA.at[:, cs], bufA0.at[:, cs],
                                   rsA_send.at[pc], rsA_recv.at[pc], right))
                rdsB.append(remote(accB.at[:, cs], bufB0.at[:, cs],
                                   rsB_send.at[pc], rsB_recv.at[pc], left))
            for r in rdsA:
                r.start()
            for r in rdsB:
                r.start()
            partial_half(bufA1, lax.rem(d - s - 1 + N_DEV, N_DEV), 0)
            partial_half(bufB1, lax.rem(d + s + 1, N_DEV), 1)
            mult = scale if s == N_DEV - 2 else None
            for pc in range(P):
                cs = pl.ds(pc * pn, pn)
                rdsA[pc].wait()
                pA = (bufA1[:, cs].astype(jnp.float32)
                      + bufA0[:, cs].astype(jnp.float32))
                accA[:, cs] = (pA if mult is None else pA * mult
                               ).astype(jnp.bfloat16)
                rdsB[pc].wait()
                pB = (bufB1[:, cs].astype(jnp.float32)
                      + bufB0[:, cs].astype(jnp.float32))
                accB[:, cs] = (pB if mult is None else pB * mult
                               ).astype(jnp.bfloat16)
            if s < N_DEV - 2:
                tA = token(rtokA_s, rtokA_r, left)
                tB = token(rtokB_s, rtokB_r, right)
                tA.start()
                tB.start()
                tA.wait_send()
                tB.wait_send()

        ownA = lax.rem(d + 1, N_DEV)
        ownB = lax.rem(d - 1 + N_DEV, N_DEV)
        cpA = pltpu.make_async_copy(
            accA, out_ref.at[pl.ds(ownA * ch, ch), pl.ds(0, hn)], own_semA)
        cpB = pltpu.make_async_copy(
            accB, out_ref.at[pl.ds(ownB * ch, ch), pl.ds(hn, hn)], own_semB)
        cpA.start()
        cpB.start()

        slotsA = (bufA0, bufA1)
        slotsB = (bufB0, bufB1)
        out_cps = {}
        for t in range(N_DEV - 1):
            if t >= 2:
                token(atokA_s, atokA_r, left).wait_recv()
                token(atokB_s, atokB_r, right).wait_recv()
            srcA = accA if t == 0 else slotsA[(t - 1) % 2]
            srcB = accB if t == 0 else slotsB[(t - 1) % 2]
            rdA = remote(srcA, slotsA[t % 2],
                         agA_send.at[t % 2], agA_recv.at[t % 2], right)
            rdB = remote(srcB, slotsB[t % 2],
                         agB_send.at[t % 2], agB_recv.at[t % 2], left)
            rdA.start()
            rdB.start()
            rdA.wait()
            rdB.wait()
            rowA = lax.rem(d - t + N_DEV, N_DEV)
            rowB = lax.rem(d + t, N_DEV)
            cpA = pltpu.make_async_copy(
                slotsA[t % 2],
                out_ref.at[pl.ds(rowA * ch, ch), pl.ds(0, hn)],
                copyA_sems.at[t % 2])
            cpB = pltpu.make_async_copy(
                slotsB[t % 2],
                out_ref.at[pl.ds(rowB * ch, ch), pl.ds(hn, hn)],
                copyB_sems.at[t % 2])
            cpA.start()
            cpB.start()
            out_cps[t % 2] = (cpA, cpB)
            if 1 <= t <= N_DEV - 3:
                pA, pB = out_cps[(t - 1) % 2]
                pA.wait()
                pB.wait()
                tA = token(atokA_s, atokA_r, left)
                tB = token(atokB_s, atokB_r, right)
                tA.start()
                tB.start()
                tA.wait_send()
                tB.wait_send()

        for slot in ((N_DEV - 3) % 2, (N_DEV - 2) % 2):
            pA, pB = out_cps[slot]
            pA.wait()
            pB.wait()
        pltpu.make_async_copy(
            accA, out_ref.at[pl.ds(ownA * ch, ch), pl.ds(0, hn)],
            own_semA).wait()
        pltpu.make_async_copy(
            accB, out_ref.at[pl.ds(ownB * ch, ch), pl.ds(hn, hn)],
            own_semB).wait()

    return pl.pallas_call(
        body,
        out_shape=jax.ShapeDtypeStruct((m, n), jnp.bfloat16),
        in_specs=[
            pl.BlockSpec(memory_space=pl.ANY),
            pl.BlockSpec(memory_space=pl.ANY),
            pl.BlockSpec(memory_space=pltpu.SMEM),
            pl.BlockSpec(memory_space=pltpu.SMEM),
        ],
        out_specs=pl.BlockSpec(memory_space=pl.ANY),
        scratch_shapes=[
            pltpu.VMEM((m, k), jnp.bfloat16),
            pltpu.VMEM((k, n), jnp.bfloat16),
            pltpu.VMEM((ch, hn), jnp.bfloat16),
            pltpu.VMEM((ch, hn), jnp.bfloat16),
            pltpu.VMEM((ch, hn), jnp.bfloat16),
            pltpu.VMEM((ch, hn), jnp.bfloat16),
            pltpu.VMEM((ch, hn), jnp.bfloat16),
            pltpu.VMEM((ch, hn), jnp.bfloat16),
            pltpu.VMEM((ch, k), jnp.float32),
            pltpu.VMEM((ch, k), jnp.float32),
            pltpu.VMEM((k, qn), jnp.float32),
            pltpu.VMEM((k, qn), jnp.float32),
            pltpu.VMEM((8, 128), jnp.float32),
            pltpu.VMEM((8, 128), jnp.float32),
            pltpu.SemaphoreType.DMA((P,)),
            pltpu.SemaphoreType.DMA((P,)),
            pltpu.SemaphoreType.DMA((P,)),
            pltpu.SemaphoreType.DMA((P,)),
            pltpu.SemaphoreType.DMA((2,)),
            pltpu.SemaphoreType.DMA((2,)),
            pltpu.SemaphoreType.DMA((2,)),
            pltpu.SemaphoreType.DMA((2,)),
            pltpu.SemaphoreType.DMA,
            pltpu.SemaphoreType.DMA,
            pltpu.SemaphoreType.DMA,
            pltpu.SemaphoreType.DMA,
            pltpu.SemaphoreType.DMA,
            pltpu.SemaphoreType.DMA,
            pltpu.SemaphoreType.DMA,
            pltpu.SemaphoreType.DMA,
            pltpu.SemaphoreType.DMA((2,)),
            pltpu.SemaphoreType.DMA((2,)),
            pltpu.SemaphoreType.DMA,
            pltpu.SemaphoreType.DMA,
            pltpu.SemaphoreType.DMA((2,)),
        ],
        compiler_params=pltpu.CompilerParams(
            collective_id=0, vmem_limit_bytes=100 * 1024 * 1024),
    )(x, w_mat, scale_x, scale_w)
